# baseline (speedup 1.0000x reference)
import jax
import jax.numpy as jnp
from jax.experimental import pallas as pl
from jax.experimental.pallas import tpu as pltpu
from jax.experimental.pallas import tpu_sc as plsc

M = 16384
D = 256
K = 8192
HALVES = 2
HM = M // HALVES

BM = 2048
BK = 1024
BN = 512
LANES = 128
GW = 128


def _z_body(x_ref, w_ref, b_ref, z2_ref, zsq_ref):
    z = jax.lax.dot_general(
        x_ref[...], w_ref[...], (((1,), (1,)), ((), ())),
        preferred_element_type=jnp.float32)
    z = z + b_ref[...]
    zsq_ref[...] = jnp.sum(z * z, axis=1, keepdims=True)
    z2_ref[...] = z + z


def _project(xf, W, b2):
    return pl.pallas_call(
        _z_body,
        grid=(M // BM,),
        in_specs=[
            pl.BlockSpec((BM, D), lambda m: (m, 0)),
            pl.BlockSpec((D, D), lambda m: (0, 0)),
            pl.BlockSpec((1, D), lambda m: (0, 0)),
        ],
        out_specs=[
            pl.BlockSpec((BM, D), lambda m: (m, 0)),
            pl.BlockSpec((BM, 1), lambda m: (m, 0)),
        ],
        out_shape=[
            jax.ShapeDtypeStruct((M, D), jnp.float32),
            jax.ShapeDtypeStruct((M, 1), jnp.float32),
        ],
    )(xf, W, b2)


def _dist_body(z2_ref, zsq_ref, cb_ref, esq_ref,
               idx_ref, loss_ref, min_ref, arg_ref):
    k = pl.program_id(1)

    @pl.when(k == 0)
    def _():
        min_ref[...] = jnp.full((BM, LANES), jnp.inf, jnp.float32)
        arg_ref[...] = jnp.zeros((BM, LANES), jnp.int32)

    z2 = z2_ref[...]
    zsq = zsq_ref[...]
    gbase = jax.lax.broadcasted_iota(jnp.int32, (BM, LANES), 1) + k * BK
    for c in range(BK // BN):
        dot2 = jax.lax.dot_general(
            z2, cb_ref[c * BN:(c + 1) * BN, :],
            (((1,), (1,)), ((), ())),
            preferred_element_type=jnp.float32)
        accm = min_ref[...]
        acci = arg_ref[...]
        for j in range(BN // LANES):
            col0 = c * BN + j * LANES
            dj = (zsq - dot2[:, j * LANES:(j + 1) * LANES]) \
                + esq_ref[0:1, col0:col0 + LANES]
            upd = dj < accm
            accm = jnp.where(upd, dj, accm)
            acci = jnp.where(upd, gbase + col0, acci)
        min_ref[...] = accm
        arg_ref[...] = acci

    @pl.when(k == pl.num_programs(1) - 1)
    def _():
        accm = min_ref[...]
        rowmin = jnp.min(accm, axis=1, keepdims=True)
        cand = jnp.where(accm == rowmin, arg_ref[...], jnp.int32(2147483647))
        idx_ref[...] = jnp.min(cand, axis=1, keepdims=True)

        @pl.when(pl.program_id(0) == 0)
        def _():
            loss_ref[...] = jnp.zeros((1, 1), jnp.float32)

        loss_ref[...] += jnp.sum(rowmin)[None, None]


def _vq_argmin(z2, zsq, codebook, esq, half):
    moff = half * (HM // BM)
    return pl.pallas_call(
        _dist_body,
        grid=(HM // BM, K // BK),
        in_specs=[
            pl.BlockSpec((BM, D), lambda m, k: (m + moff, 0)),
            pl.BlockSpec((BM, 1), lambda m, k: (m + moff, 0)),
            pl.BlockSpec((BK, D), lambda m, k: (k, 0)),
            pl.BlockSpec((1, BK), lambda m, k: (0, k)),
        ],
        out_specs=[
            pl.BlockSpec((BM, 1), lambda m, k: (m, 0)),
            pl.BlockSpec((1, 1), lambda m, k: (0, 0)),
        ],
        out_shape=[
            jax.ShapeDtypeStruct((HM, 1), jnp.int32),
            jax.ShapeDtypeStruct((1, 1), jnp.float32),
        ],
        scratch_shapes=[
            pltpu.VMEM((BM, LANES), jnp.float32),
            pltpu.VMEM((BM, LANES), jnp.int32),
        ],
        compiler_params=pltpu.CompilerParams(
            dimension_semantics=("arbitrary", "arbitrary")),
    )(z2, zsq, codebook, esq)


def _sc_gather(codebook, idx_row):
    mesh = plsc.VectorSubcoreMesh(core_axis_name="c", subcore_axis_name="s")

    @pl.kernel(out_type=jax.ShapeDtypeStruct((HM, D), jnp.float32), mesh=mesh)
    def gather_kernel(cb_hbm, i_hbm, o_hbm):
        def body(i_vmem, o_vmem):
            pltpu.sync_copy(cb_hbm.at[i_vmem.at[0]], o_vmem)

        pltpu.emit_pipeline(
            body,
            grid=(HM // GW,),
            in_specs=[pl.BlockSpec((1, GW), index_map=lambda i: (0, i))],
            out_specs=[pl.BlockSpec((GW, D), index_map=lambda i: (i, 0))],
            core_axis_name=("c", "s"),
            dimension_semantics=(pltpu.PARALLEL,),
        )(i_hbm, o_hbm)

    return gather_kernel(codebook, idx_row)


def kernel(x, W, b, codebook):
    xf = x.reshape(M, D)
    b2 = b.reshape(1, D)
    esq = jnp.sum(codebook ** 2, axis=1)[None, :]
    z2, zsq = _project(xf, W, b2)
    loss_sum = jnp.float32(0.0)
    quant_halves = []
    for h in range(HALVES):
        idx, lsum = _vq_argmin(z2, zsq, codebook, esq, h)
        quant_halves.append(_sc_gather(codebook, idx.reshape(1, HM)))
        loss_sum = loss_sum + lsum[0, 0]
    loss = loss_sum / jnp.float32(M * D)
    x_recon = jnp.concatenate(quant_halves, axis=0).reshape(x.shape)
    return loss, loss, x_recon


# speedup vs baseline: 1.0969x; 1.0969x over previous
import jax
import jax.numpy as jnp
from jax.experimental import pallas as pl
from jax.experimental.pallas import tpu as pltpu
from jax.experimental.pallas import tpu_sc as plsc

M = 16384
D = 256
K = 8192

BM = 2048
BN = 512
LANES = 128
JC = BN // LANES
GW = 128


def _vq_body(x_ref, w_ref, b_ref, cb_ref, esq_ref,
             idx_ref, loss_ref, min_ref, arg_ref):
    z = jax.lax.dot_general(
        x_ref[...], w_ref[...], (((1,), (1,)), ((), ())),
        preferred_element_type=jnp.float32)
    z = z + b_ref[...]
    zsq = jnp.sum(z * z, axis=1, keepdims=True)
    z2 = z + z

    min_ref[...] = jnp.full((BM, LANES), jnp.inf, jnp.float32)
    arg_ref[...] = jnp.zeros((BM, LANES), jnp.int32)

    for c in range(K // BN):
        dot2 = jax.lax.dot_general(
            z2, cb_ref[c * BN:(c + 1) * BN, :],
            (((1,), (1,)), ((), ())),
            preferred_element_type=jnp.float32)
        accm = min_ref[...]
        acci = arg_ref[...]
        for j in range(JC):
            o = c * JC + j
            dj = (zsq - dot2[:, j * LANES:(j + 1) * LANES]) \
                + esq_ref[0:1, o * LANES:(o + 1) * LANES]
            upd = dj < accm
            accm = jnp.where(upd, dj, accm)
            acci = jnp.where(upd, o, acci)
        min_ref[...] = accm
        arg_ref[...] = acci

    accm = min_ref[...]
    gidx = arg_ref[...] * LANES \
        + jax.lax.broadcasted_iota(jnp.int32, (BM, LANES), 1)
    rowmin = jnp.min(accm, axis=1, keepdims=True)
    cand = jnp.where(accm == rowmin, gidx, jnp.int32(2147483647))
    idx_ref[...] = jnp.min(cand, axis=1, keepdims=True)

    @pl.when(pl.program_id(0) == 0)
    def _():
        loss_ref[...] = jnp.zeros((1, 1), jnp.float32)

    loss_ref[...] += jnp.sum(rowmin)[None, None]


def _vq_argmin(xf, W, b2, codebook, esq):
    return pl.pallas_call(
        _vq_body,
        grid=(M // BM,),
        in_specs=[
            pl.BlockSpec((BM, D), lambda m: (m, 0)),
            pl.BlockSpec((D, D), lambda m: (0, 0)),
            pl.BlockSpec((1, D), lambda m: (0, 0)),
            pl.BlockSpec((K, D), lambda m: (0, 0)),
            pl.BlockSpec((1, K), lambda m: (0, 0)),
        ],
        out_specs=[
            pl.BlockSpec((BM, 1), lambda m: (m, 0)),
            pl.BlockSpec((1, 1), lambda m: (0, 0)),
        ],
        out_shape=[
            jax.ShapeDtypeStruct((M, 1), jnp.int32),
            jax.ShapeDtypeStruct((1, 1), jnp.float32),
        ],
        scratch_shapes=[
            pltpu.VMEM((BM, LANES), jnp.float32),
            pltpu.VMEM((BM, LANES), jnp.int32),
        ],
        compiler_params=pltpu.CompilerParams(
            dimension_semantics=("arbitrary",)),
    )(xf, W, b2, codebook, esq)


def _sc_gather(codebook, idx_row):
    mesh = plsc.VectorSubcoreMesh(core_axis_name="c", subcore_axis_name="s")

    @pl.kernel(out_type=jax.ShapeDtypeStruct((M, D), jnp.float32), mesh=mesh)
    def gather_kernel(cb_hbm, i_hbm, o_hbm):
        def body(i_vmem, o_vmem):
            pltpu.sync_copy(cb_hbm.at[i_vmem.at[0]], o_vmem)

        pltpu.emit_pipeline(
            body,
            grid=(M // GW,),
            in_specs=[pl.BlockSpec((1, GW), index_map=lambda i: (0, i))],
            out_specs=[pl.BlockSpec((GW, D), index_map=lambda i: (i, 0))],
            core_axis_name=("c", "s"),
            dimension_semantics=(pltpu.PARALLEL,),
        )(i_hbm, o_hbm)

    return gather_kernel(codebook, idx_row)


def kernel(x, W, b, codebook):
    xf = x.reshape(M, D)
    b2 = b.reshape(1, D)
    esq = jnp.sum(codebook ** 2, axis=1)[None, :]
    idx, loss_sum = _vq_argmin(xf, W, b2, codebook, esq)
    quantized = _sc_gather(codebook, idx.reshape(1, M))
    loss = loss_sum[0, 0] / jnp.float32(M * D)
    x_recon = quantized.reshape(x.shape)
    return loss, loss, x_recon


# speedup vs baseline: 1.1043x; 1.0067x over previous
import jax
import jax.numpy as jnp
from jax.experimental import pallas as pl
from jax.experimental.pallas import tpu as pltpu
from jax.experimental.pallas import tpu_sc as plsc

M = 16384
D = 256
K = 8192

BM = 2048
BN = 512
LANES = 128
JC = BN // LANES
RG = 256
GW = 128


def _vq_body(x_ref, w_ref, b_ref, cb_ref, esq_ref,
             idx_ref, loss_ref, min_ref, arg_ref):
    z = jax.lax.dot_general(
        x_ref[...], w_ref[...], (((1,), (1,)), ((), ())),
        preferred_element_type=jnp.float32)
    z = z + b_ref[...]
    zsq = jnp.sum(z * z, axis=1, keepdims=True)
    z2 = z + z

    min_ref[...] = jnp.full((BM, LANES), jnp.inf, jnp.float32)
    arg_ref[...] = jnp.zeros((BM, LANES), jnp.int32)

    for c in range(K // BN):
        dot2 = jax.lax.dot_general(
            z2, cb_ref[c * BN:(c + 1) * BN, :],
            (((1,), (1,)), ((), ())),
            preferred_element_type=jnp.float32)
        for r in range(BM // RG):
            rs = slice(r * RG, (r + 1) * RG)
            accm = min_ref[rs, :]
            acci = arg_ref[rs, :]
            zsq_r = zsq[r * RG:(r + 1) * RG, :]
            for j in range(JC):
                o = c * JC + j
                dj = (zsq_r - dot2[r * RG:(r + 1) * RG,
                                   j * LANES:(j + 1) * LANES]) \
                    + esq_ref[0:1, o * LANES:(o + 1) * LANES]
                upd = dj < accm
                accm = jnp.where(upd, dj, accm)
                acci = jnp.where(upd, o, acci)
            min_ref[rs, :] = accm
            arg_ref[rs, :] = acci

    accm = min_ref[...]
    gidx = arg_ref[...] * LANES \
        + jax.lax.broadcasted_iota(jnp.int32, (BM, LANES), 1)
    rowmin = jnp.min(accm, axis=1, keepdims=True)
    cand = jnp.where(accm == rowmin, gidx, jnp.int32(2147483647))
    idx_ref[...] = jnp.min(cand, axis=1, keepdims=True)

    @pl.when(pl.program_id(0) == 0)
    def _():
        loss_ref[...] = jnp.zeros((1, 1), jnp.float32)

    loss_ref[...] += jnp.sum(rowmin)[None, None]


def _vq_argmin(xf, W, b2, codebook, esq):
    return pl.pallas_call(
        _vq_body,
        grid=(M // BM,),
        in_specs=[
            pl.BlockSpec((BM, D), lambda m: (m, 0)),
            pl.BlockSpec((D, D), lambda m: (0, 0)),
            pl.BlockSpec((1, D), lambda m: (0, 0)),
            pl.BlockSpec((K, D), lambda m: (0, 0)),
            pl.BlockSpec((1, K), lambda m: (0, 0)),
        ],
        out_specs=[
            pl.BlockSpec((BM, 1), lambda m: (m, 0)),
            pl.BlockSpec((1, 1), lambda m: (0, 0)),
        ],
        out_shape=[
            jax.ShapeDtypeStruct((M, 1), jnp.int32),
            jax.ShapeDtypeStruct((1, 1), jnp.float32),
        ],
        scratch_shapes=[
            pltpu.VMEM((BM, LANES), jnp.float32),
            pltpu.VMEM((BM, LANES), jnp.int32),
        ],
        compiler_params=pltpu.CompilerParams(
            dimension_semantics=("arbitrary",)),
    )(xf, W, b2, codebook, esq)


def _sc_gather(codebook, idx_row):
    mesh = plsc.VectorSubcoreMesh(core_axis_name="c", subcore_axis_name="s")

    @pl.kernel(out_type=jax.ShapeDtypeStruct((M, D), jnp.float32), mesh=mesh)
    def gather_kernel(cb_hbm, i_hbm, o_hbm):
        def body(i_vmem, o_vmem):
            pltpu.sync_copy(cb_hbm.at[i_vmem.at[0]], o_vmem)

        pltpu.emit_pipeline(
            body,
            grid=(M // GW,),
            in_specs=[pl.BlockSpec((1, GW), index_map=lambda i: (0, i))],
            out_specs=[pl.BlockSpec((GW, D), index_map=lambda i: (i, 0))],
            core_axis_name=("c", "s"),
            dimension_semantics=(pltpu.PARALLEL,),
        )(i_hbm, o_hbm)

    return gather_kernel(codebook, idx_row)


def kernel(x, W, b, codebook):
    xf = x.reshape(M, D)
    b2 = b.reshape(1, D)
    esq = jnp.sum(codebook ** 2, axis=1)[None, :]
    idx, loss_sum = _vq_argmin(xf, W, b2, codebook, esq)
    quantized = _sc_gather(codebook, idx.reshape(1, M))
    loss = loss_sum[0, 0] / jnp.float32(M * D)
    x_recon = quantized.reshape(x.shape)
    return loss, loss, x_recon
